# baseline (speedup 1.0000x reference)
import functools

import jax
import jax.numpy as jnp
from jax import lax
from jax.experimental import pallas as pl
from jax.experimental.pallas import tpu as pltpu
from jax.experimental.pallas import tpu_sc as plsc

N = 10000
E = 320000
D = 128
G = 16
NPG = N // G
NC = 2
NS = 16
NW = NC * NS
EPW = E // NW
EPT = E // NS
CH = 64
NCH = 160
EPWP = NCH * CH
NBUF = 4
NPAD = 10240
RED = NPAD // NS
NPADM = 10112
REDM = NPADM // NS


def _deg_body(sr_hbm, out_hbm, idx_v, cnt_v, redbuf, outbuf, shared):
    cid = lax.axis_index("c")
    sid = lax.axis_index("s")
    zeros = jnp.zeros((16,), jnp.float32)
    ones = jnp.ones((16,), jnp.float32)

    def zbody(i, _):
        cnt_v[pl.ds(i * 16, 16)] = zeros
        return 0
    lax.fori_loop(0, NPAD // 16, zbody, 0)

    pltpu.sync_copy(sr_hbm.at[pl.ds(cid * E + sid * EPT, EPT)], idx_v)

    def cbody(i, _):
        si = idx_v[pl.ds(i * 16, 16)]
        plsc.addupdate_scatter(cnt_v, [si], ones)
        return 0
    lax.fori_loop(0, EPT // 16, cbody, 0)

    pltpu.sync_copy(cnt_v, shared.at[sid])
    plsc.subcore_barrier()

    lo = sid * RED
    pltpu.sync_copy(shared.at[:, pl.ds(lo, RED)], redbuf)

    def rbody(i, _):
        acc = redbuf[0, pl.ds(i * 16, 16)]
        for t in range(1, NS):
            acc = acc + redbuf[t, pl.ds(i * 16, 16)]
        outbuf[pl.ds(i * 16, 16)] = acc
        return 0
    lax.fori_loop(0, RED // 16, rbody, 0)

    pltpu.sync_copy(outbuf, out_hbm.at[cid, pl.ds(lo, RED)])


def _degree_counts(senders, receivers):
    mesh = plsc.VectorSubcoreMesh(core_axis_name="c", subcore_axis_name="s")
    return pl.kernel(
        _deg_body,
        compiler_params=pltpu.CompilerParams(needs_layout_passes=False),
        out_type=jax.ShapeDtypeStruct((2, NPAD), jnp.float32),
        mesh=mesh,
        scratch_types=[
            pltpu.VMEM((EPT,), jnp.int32),
            pltpu.VMEM((NPAD,), jnp.float32),
            pltpu.VMEM((NS, RED), jnp.float32),
            pltpu.VMEM((RED,), jnp.float32),
            pltpu.VMEM_SHARED((NS, NPAD), jnp.float32),
        ],
    )(jnp.concatenate([senders, receivers]))


def _msg_body(m_hbm, pidx_hbm, out_hbm, pidx_v, sc0, rc0, sc1, rc1, sc2, rc2,
              sc3, rc3, rows0, rows1, rows2, rows3, zbuf, acc,
              g0, g1, g2, g3, t0, t1, t2, t3):
    cid = lax.axis_index("c")
    sid = lax.axis_index("s")
    w = cid * NS + sid

    pltpu.sync_copy(pidx_hbm.at[w], pidx_v)

    zeros = jnp.zeros((16,), jnp.float32)

    def zrow(r, _):
        for c in range(D // 16):
            zbuf[r, pl.ds(c * 16, 16)] = zeros
        return 0
    lax.fori_loop(0, 8, zrow, 0)

    def zcopy(j, _):
        pltpu.sync_copy(zbuf, acc.at[pl.ds(sid * REDM + j * 8, 8)])
        return 0
    lax.fori_loop(0, REDM // 8, zcopy, 0)
    plsc.subcore_barrier()

    scs = [sc0, sc1, sc2, sc3]
    rcs = [rc0, rc1, rc2, rc3]
    rows = [rows0, rows1, rows2, rows3]
    gs = [g0, g1, g2, g3]
    ts = [t0, t1, t2, t3]

    def unpack(i, b):
        def ub(j, _):
            v = pidx_v[pl.ds(i * CH + j * 16, 16)]
            scs[b][pl.ds(j * 16, 16)] = v & 0xFFFF
            rcs[b][pl.ds(j * 16, 16)] = v >> 16
            return 0
        lax.fori_loop(0, CH // 16, ub, 0)

    def start_gather(i, b):
        unpack(i, b)
        pltpu.async_copy(m_hbm.at[scs[b]], rows[b], gs[b])

    def wait_gather(b):
        pltpu.make_async_copy(m_hbm.at[scs[b]], rows[b], gs[b]).wait()

    def start_scatter(b):
        pltpu.async_copy(rows[b], acc.at[rcs[b]], ts[b], add=True)

    def wait_scatter(b):
        pltpu.make_async_copy(rows[b], acc.at[rcs[b]], ts[b]).wait()

    for c in range(NBUF):
        start_gather(c, c)
    wait_gather(0)
    start_scatter(0)

    def pbody(q, _):
        for j in range(NBUF):
            c = NBUF * q + 1 + j
            b = (1 + j) % NBUF
            bl = j
            wait_gather(b)
            start_scatter(b)
            wait_scatter(bl)
            start_gather(c + NBUF - 1, bl)
        return 0
    lax.fori_loop(0, (NCH - NBUF) // NBUF, pbody, 0)

    for c in range(NCH - 3, NCH):
        b = c % NBUF
        wait_gather(b)
        start_scatter(b)
        wait_scatter((c - 1) % NBUF)
    wait_scatter((NCH - 1) % NBUF)

    plsc.subcore_barrier()
    pltpu.sync_copy(acc.at[pl.ds(sid * REDM, REDM)],
                    out_hbm.at[cid, pl.ds(sid * REDM, REDM)])


def _messages(m, pidx2):
    mesh = plsc.VectorSubcoreMesh(core_axis_name="c", subcore_axis_name="s")
    return pl.kernel(
        _msg_body,
        out_type=jax.ShapeDtypeStruct((NC, NPADM, D), jnp.float32),
        mesh=mesh,
        scratch_types=[
            pltpu.VMEM((EPWP,), jnp.int32),
            pltpu.VMEM((CH,), jnp.int32),
            pltpu.VMEM((CH,), jnp.int32),
            pltpu.VMEM((CH,), jnp.int32),
            pltpu.VMEM((CH,), jnp.int32),
            pltpu.VMEM((CH,), jnp.int32),
            pltpu.VMEM((CH,), jnp.int32),
            pltpu.VMEM((CH,), jnp.int32),
            pltpu.VMEM((CH,), jnp.int32),
            pltpu.VMEM((CH, D), jnp.float32),
            pltpu.VMEM((CH, D), jnp.float32),
            pltpu.VMEM((CH, D), jnp.float32),
            pltpu.VMEM((CH, D), jnp.float32),
            pltpu.VMEM((8, D), jnp.float32),
            pltpu.VMEM_SHARED((NPADM, D), jnp.float32),
            pltpu.SemaphoreType.DMA,
            pltpu.SemaphoreType.DMA,
            pltpu.SemaphoreType.DMA,
            pltpu.SemaphoreType.DMA,
            pltpu.SemaphoreType.DMA,
            pltpu.SemaphoreType.DMA,
            pltpu.SemaphoreType.DMA,
            pltpu.SemaphoreType.DMA,
        ],
    )(m, pidx2)


ROWS = 1000
NBLK = N // ROWS


def _layer_norm(x, scale, bias, eps=1e-6):
    mean = jnp.mean(x, axis=-1, keepdims=True)
    var = jnp.mean(jnp.square(x - mean), axis=-1, keepdims=True)
    return (x - mean) * lax.rsqrt(var + eps) * scale + bias


def _mlp_block(x, W1, b1, g1, be1, W2, b2, g2, be2, W3, b3):
    x = jax.nn.relu(jnp.dot(x, W1, preferred_element_type=jnp.float32) + b1)
    x = _layer_norm(x, g1, be1)
    x = jax.nn.relu(jnp.dot(x, W2, preferred_element_type=jnp.float32) + b2)
    x = _layer_norm(x, g2, be2)
    return jax.nn.relu(jnp.dot(x, W3, preferred_element_type=jnp.float32) + b3)


def _embed_mlp_body(nodes, We, be, W1, b1, g1, be1, W2, b2, g2, be2, W3, b3,
                    scnt, out):
    h = jnp.dot(nodes[...], We[...], preferred_element_type=jnp.float32) + be[...]
    x = _mlp_block(h, W1[...], b1[...], g1[...], be1[...], W2[...], b2[...],
                   g2[...], be2[...], W3[...], b3[...])
    out[...] = x * lax.rsqrt(scnt[...] + 1.0)


def _mid_body(p, m, rcnt, W1, b1, g1, be1, W2, b2, g2, be2, W3, b3,
              scnt, out):
    h = (p[0] + p[1] + m[...]) * lax.rsqrt(rcnt[...] + 1.0)
    x = _mlp_block(h, W1[...], b1[...], g1[...], be1[...], W2[...], b2[...],
                   g2[...], be2[...], W3[...], b3[...])
    out[...] = x * lax.rsqrt(scnt[...] + 1.0)


def _final_body(p, m, rcnt, Wd, bd, out):
    h = (p[0] + p[1] + m[...]) * lax.rsqrt(rcnt[...] + 1.0)
    gids = lax.broadcasted_iota(jnp.int32, (G, N), 0)
    nids = lax.broadcasted_iota(jnp.int32, (G, N), 1) // NPG
    seg = jnp.where(gids == nids, 1.0 / NPG, 0.0)
    pooled = jnp.dot(seg, h, preferred_element_type=jnp.float32)
    out[...] = jnp.dot(pooled, Wd[...], preferred_element_type=jnp.float32) + bd[...]


def _full(shape):
    return pl.BlockSpec(shape, lambda i: (0,) * len(shape))


def _wspecs():
    return [_full((D, D)), _full((1, D)), _full((1, D)), _full((1, D)),
            _full((D, D)), _full((1, D)), _full((1, D)), _full((1, D)),
            _full((D, D)), _full((1, D))]


def _embed_mlp(nodes, We, be, ws, scnt):
    return pl.pallas_call(
        _embed_mlp_body,
        grid=(NBLK,),
        in_specs=[pl.BlockSpec((ROWS, D), lambda i: (i, 0)),
                  _full((D, D)), _full((1, D)), *_wspecs(),
                  pl.BlockSpec((ROWS, 1), lambda i: (i, 0))],
        out_specs=pl.BlockSpec((ROWS, D), lambda i: (i, 0)),
        out_shape=jax.ShapeDtypeStruct((N, D), jnp.float32),
    )(nodes, We, be, *ws, scnt)


def _mid(p, m, rcnt, ws, scnt):
    return pl.pallas_call(
        _mid_body,
        grid=(NBLK,),
        in_specs=[pl.BlockSpec((NC, ROWS, D), lambda i: (0, i, 0)),
                  pl.BlockSpec((ROWS, D), lambda i: (i, 0)),
                  pl.BlockSpec((ROWS, 1), lambda i: (i, 0)),
                  *_wspecs(),
                  pl.BlockSpec((ROWS, 1), lambda i: (i, 0))],
        out_specs=pl.BlockSpec((ROWS, D), lambda i: (i, 0)),
        out_shape=jax.ShapeDtypeStruct((N, D), jnp.float32),
    )(p, m, rcnt, *ws, scnt)


def _final(p, m, rcnt, Wd, bd):
    return pl.pallas_call(
        _final_body,
        grid=(1,),
        in_specs=[pl.BlockSpec((NC, N, D), lambda i: (0, 0, 0)),
                  pl.BlockSpec((N, D), lambda i: (0, 0)),
                  pl.BlockSpec((N, 1), lambda i: (0, 0)),
                  pl.BlockSpec((D, D), lambda i: (0, 0)),
                  pl.BlockSpec((1, D), lambda i: (0, 0))],
        out_specs=pl.BlockSpec((G, D), lambda i: (0, 0)),
        out_shape=jax.ShapeDtypeStruct((G, D), jnp.float32),
    )(p, m, rcnt, Wd, bd)


def kernel(nodes, edges, senders, receivers, globals_, n_node, n_edge,
           W_embed, b_embed, mlp_W1, mlp_b1, ln1_scale, ln1_bias,
           mlp_W2, mlp_b2, ln2_scale, ln2_bias, mlp_W3, mlp_b3,
           W_dec, b_dec):
    senders = senders.astype(jnp.int32)
    receivers = receivers.astype(jnp.int32)

    cnt = _degree_counts(senders, receivers)
    scnt = cnt[0, :N, None]
    rcnt = cnt[1, :N, None]

    packed = senders | (receivers << 16)
    fill = jnp.full((NW * EPWP - E,), N << 16, dtype=jnp.int32)
    pidx2 = jnp.concatenate([packed, fill]).reshape(NW, EPWP)

    def ws(i):
        return [mlp_W1[i], mlp_b1[i][None], ln1_scale[i][None],
                ln1_bias[i][None], mlp_W2[i], mlp_b2[i][None],
                ln2_scale[i][None], ln2_bias[i][None], mlp_W3[i],
                mlp_b3[i][None]]

    m0 = _embed_mlp(nodes, W_embed, b_embed[None], ws(0), scnt)
    p0 = _messages(m0, pidx2)
    m1 = _mid(p0, m0, rcnt, ws(1), scnt)
    p1 = _messages(m1, pidx2)
    return _final(p1, m1, rcnt, W_dec, b_dec[None])


# speedup vs baseline: 1.0795x; 1.0795x over previous
import functools

import jax
import jax.numpy as jnp
from jax import lax
from jax.experimental import pallas as pl
from jax.experimental.pallas import tpu as pltpu
from jax.experimental.pallas import tpu_sc as plsc

N = 10000
E = 320000
D = 128
G = 16
NPG = N // G
NC = 2
NS = 16
NW = NC * NS
EPW = E // NW
EPT = E // NS
CH = 64
NCH = 160
EPWP = NCH * CH
NBUF = 4
NPAD = 10240
RED = NPAD // NS
NPADM = 10112
REDM = NPADM // NS


def _deg_body(sr_hbm, out_hbm, idx_v, cnt_v, redbuf, outbuf, shared):
    cid = lax.axis_index("c")
    sid = lax.axis_index("s")
    zeros = jnp.zeros((16,), jnp.float32)
    ones = jnp.ones((16,), jnp.float32)

    def zbody(i, _):
        cnt_v[pl.ds(i * 16, 16)] = zeros
        return 0
    lax.fori_loop(0, NPAD // 16, zbody, 0)

    pltpu.sync_copy(sr_hbm.at[pl.ds(cid * E + sid * EPT, EPT)], idx_v)

    def cbody(i, _):
        si = idx_v[pl.ds(i * 16, 16)]
        plsc.addupdate_scatter(cnt_v, [si], ones)
        return 0
    lax.fori_loop(0, EPT // 16, cbody, 0)

    pltpu.sync_copy(cnt_v, shared.at[sid])
    plsc.subcore_barrier()

    lo = sid * RED
    pltpu.sync_copy(shared.at[:, pl.ds(lo, RED)], redbuf)

    def rbody(i, _):
        acc = redbuf[0, pl.ds(i * 16, 16)]
        for t in range(1, NS):
            acc = acc + redbuf[t, pl.ds(i * 16, 16)]
        outbuf[pl.ds(i * 16, 16)] = acc
        return 0
    lax.fori_loop(0, RED // 16, rbody, 0)

    pltpu.sync_copy(outbuf, out_hbm.at[cid, pl.ds(lo, RED)])


def _degree_counts(senders, receivers):
    mesh = plsc.VectorSubcoreMesh(core_axis_name="c", subcore_axis_name="s")
    return pl.kernel(
        _deg_body,
        compiler_params=pltpu.CompilerParams(needs_layout_passes=False),
        out_type=jax.ShapeDtypeStruct((2, NPAD), jnp.float32),
        mesh=mesh,
        scratch_types=[
            pltpu.VMEM((EPT,), jnp.int32),
            pltpu.VMEM((NPAD,), jnp.float32),
            pltpu.VMEM((NS, RED), jnp.float32),
            pltpu.VMEM((RED,), jnp.float32),
            pltpu.VMEM_SHARED((NS, NPAD), jnp.float32),
        ],
    )(jnp.concatenate([senders, receivers]))


def _msg_body(m_hbm, pidx_hbm, out_hbm, pidx_v, sc0, rc0, sc1, rc1, sc2, rc2,
              sc3, rc3, rows0, rows1, rows2, rows3, zbuf, acc,
              g0, g1, g2, g3, t0, t1, t2, t3):
    cid = lax.axis_index("c")
    sid = lax.axis_index("s")
    w = cid * NS + sid

    pltpu.sync_copy(pidx_hbm.at[w], pidx_v)

    zeros = jnp.zeros((16,), jnp.float32)

    def zrow(r, _):
        for c in range(D // 16):
            zbuf[r, pl.ds(c * 16, 16)] = zeros
        return 0
    lax.fori_loop(0, 8, zrow, 0)

    def zcopy(j, _):
        pltpu.sync_copy(zbuf, acc.at[pl.ds(sid * REDM + j * 8, 8)])
        return 0
    lax.fori_loop(0, REDM // 8, zcopy, 0)
    plsc.subcore_barrier()

    scs = [sc0, sc1, sc2, sc3]
    rcs = [rc0, rc1, rc2, rc3]
    rows = [rows0, rows1, rows2, rows3]
    gs = [g0, g1, g2, g3]
    ts = [t0, t1, t2, t3]

    def unpack(i, b):
        def ub(j, _):
            v = pidx_v[pl.ds(i * CH + j * 16, 16)]
            scs[b][pl.ds(j * 16, 16)] = v & 0xFFFF
            rcs[b][pl.ds(j * 16, 16)] = v >> 16
            return 0
        lax.fori_loop(0, CH // 16, ub, 0)

    def start_gather(i, b):
        unpack(i, b)
        pltpu.async_copy(m_hbm.at[scs[b]], rows[b], gs[b])

    def wait_gather(b):
        pltpu.make_async_copy(m_hbm.at[scs[b]], rows[b], gs[b]).wait()

    def start_scatter(b):
        pltpu.async_copy(rows[b], acc.at[rcs[b]], ts[b], add=True)

    def wait_scatter(b):
        pltpu.make_async_copy(rows[b], acc.at[rcs[b]], ts[b]).wait()

    for c in range(NBUF):
        start_gather(c, c)
    wait_gather(0)
    start_scatter(0)

    def pbody(q, _):
        for j in range(NBUF):
            c = NBUF * q + 1 + j
            b = (1 + j) % NBUF
            bl = j
            wait_gather(b)
            start_scatter(b)
            wait_scatter(bl)
            start_gather(c + NBUF - 1, bl)
        return 0
    lax.fori_loop(0, (NCH - NBUF) // NBUF, pbody, 0)

    for c in range(NCH - 3, NCH):
        b = c % NBUF
        wait_gather(b)
        start_scatter(b)
        wait_scatter((c - 1) % NBUF)
    wait_scatter((NCH - 1) % NBUF)

    plsc.subcore_barrier()
    pltpu.sync_copy(acc.at[pl.ds(sid * REDM, REDM)],
                    out_hbm.at[cid, pl.ds(sid * REDM, REDM)])


def _messages(m, pidx2):
    mesh = plsc.VectorSubcoreMesh(core_axis_name="c", subcore_axis_name="s")
    return pl.kernel(
        _msg_body,
        out_type=jax.ShapeDtypeStruct((NC, NPADM, D), jnp.float32),
        mesh=mesh,
        scratch_types=[
            pltpu.VMEM((EPWP,), jnp.int32),
            pltpu.VMEM((CH,), jnp.int32),
            pltpu.VMEM((CH,), jnp.int32),
            pltpu.VMEM((CH,), jnp.int32),
            pltpu.VMEM((CH,), jnp.int32),
            pltpu.VMEM((CH,), jnp.int32),
            pltpu.VMEM((CH,), jnp.int32),
            pltpu.VMEM((CH,), jnp.int32),
            pltpu.VMEM((CH,), jnp.int32),
            pltpu.VMEM((CH, D), jnp.float32),
            pltpu.VMEM((CH, D), jnp.float32),
            pltpu.VMEM((CH, D), jnp.float32),
            pltpu.VMEM((CH, D), jnp.float32),
            pltpu.VMEM((8, D), jnp.float32),
            pltpu.VMEM_SHARED((NPADM, D), jnp.float32),
            pltpu.SemaphoreType.DMA,
            pltpu.SemaphoreType.DMA,
            pltpu.SemaphoreType.DMA,
            pltpu.SemaphoreType.DMA,
            pltpu.SemaphoreType.DMA,
            pltpu.SemaphoreType.DMA,
            pltpu.SemaphoreType.DMA,
            pltpu.SemaphoreType.DMA,
        ],
    )(m, pidx2)


ROWS = 1000
NBLK = N // ROWS


def _layer_norm(x, scale, bias, eps=1e-6):
    mean = jnp.mean(x, axis=-1, keepdims=True)
    var = jnp.mean(jnp.square(x - mean), axis=-1, keepdims=True)
    return (x - mean) * lax.rsqrt(var + eps) * scale + bias


def _mlp_block(x, W1, b1, g1, be1, W2, b2, g2, be2, W3, b3):
    x = jax.nn.relu(jnp.dot(x, W1, preferred_element_type=jnp.float32) + b1)
    x = _layer_norm(x, g1, be1)
    x = jax.nn.relu(jnp.dot(x, W2, preferred_element_type=jnp.float32) + b2)
    x = _layer_norm(x, g2, be2)
    return jax.nn.relu(jnp.dot(x, W3, preferred_element_type=jnp.float32) + b3)


def _embed_mlp_body(nodes, We, be, W1, b1, g1, be1, W2, b2, g2, be2, W3, b3,
                    scnt, out):
    h = jnp.dot(nodes[...], We[...], preferred_element_type=jnp.float32) + be[...]
    x = _mlp_block(h, W1[...], b1[...], g1[...], be1[...], W2[...], b2[...],
                   g2[...], be2[...], W3[...], b3[...])
    out[...] = x * lax.rsqrt(scnt[...] + 1.0)


def _mid_body(p, m, rcnt, W1, b1, g1, be1, W2, b2, g2, be2, W3, b3,
              scnt, out):
    h = (p[0] + p[1] + m[...]) * lax.rsqrt(rcnt[...] + 1.0)
    x = _mlp_block(h, W1[...], b1[...], g1[...], be1[...], W2[...], b2[...],
                   g2[...], be2[...], W3[...], b3[...])
    out[...] = x * lax.rsqrt(scnt[...] + 1.0)


def _final_body(p, m, rcnt, Wd, bd, out):
    h = (p[0] + p[1] + m[...]) * lax.rsqrt(rcnt[...] + 1.0)
    gids = lax.broadcasted_iota(jnp.int32, (G, N), 0)
    nids = lax.broadcasted_iota(jnp.int32, (G, N), 1) // NPG
    seg = jnp.where(gids == nids, 1.0 / NPG, 0.0)
    pooled = jnp.dot(seg, h, preferred_element_type=jnp.float32)
    out[...] = jnp.dot(pooled, Wd[...], preferred_element_type=jnp.float32) + bd[...]


def _full(shape):
    return pl.BlockSpec(shape, lambda i: (0,) * len(shape))


def _wspecs():
    return [_full((D, D)), _full((1, D)), _full((1, D)), _full((1, D)),
            _full((D, D)), _full((1, D)), _full((1, D)), _full((1, D)),
            _full((D, D)), _full((1, D))]


def _embed_mlp(nodes, We, be, ws, scnt):
    return pl.pallas_call(
        _embed_mlp_body,
        grid=(NBLK,),
        in_specs=[pl.BlockSpec((ROWS, D), lambda i: (i, 0)),
                  _full((D, D)), _full((1, D)), *_wspecs(),
                  pl.BlockSpec((ROWS, 1), lambda i: (i, 0))],
        out_specs=pl.BlockSpec((ROWS, D), lambda i: (i, 0)),
        out_shape=jax.ShapeDtypeStruct((N, D), jnp.float32),
    )(nodes, We, be, *ws, scnt)


def _mid(p, m, rcnt, ws, scnt):
    return pl.pallas_call(
        _mid_body,
        grid=(NBLK,),
        in_specs=[pl.BlockSpec((NC, ROWS, D), lambda i: (0, i, 0)),
                  pl.BlockSpec((ROWS, D), lambda i: (i, 0)),
                  pl.BlockSpec((ROWS, 1), lambda i: (i, 0)),
                  *_wspecs(),
                  pl.BlockSpec((ROWS, 1), lambda i: (i, 0))],
        out_specs=pl.BlockSpec((ROWS, D), lambda i: (i, 0)),
        out_shape=jax.ShapeDtypeStruct((N, D), jnp.float32),
    )(p, m, rcnt, *ws, scnt)


def _final(p, m, rcnt, Wd, bd):
    return pl.pallas_call(
        _final_body,
        grid=(1,),
        in_specs=[pl.BlockSpec((NC, N, D), lambda i: (0, 0, 0)),
                  pl.BlockSpec((N, D), lambda i: (0, 0)),
                  pl.BlockSpec((N, 1), lambda i: (0, 0)),
                  pl.BlockSpec((D, D), lambda i: (0, 0)),
                  pl.BlockSpec((1, D), lambda i: (0, 0))],
        out_specs=pl.BlockSpec((G, D), lambda i: (0, 0)),
        out_shape=jax.ShapeDtypeStruct((G, D), jnp.float32),
    )(p, m, rcnt, Wd, bd)


def kernel(nodes, edges, senders, receivers, globals_, n_node, n_edge,
           W_embed, b_embed, mlp_W1, mlp_b1, ln1_scale, ln1_bias,
           mlp_W2, mlp_b2, ln2_scale, ln2_bias, mlp_W3, mlp_b3,
           W_dec, b_dec):
    senders = senders.astype(jnp.int32)
    receivers = receivers.astype(jnp.int32)

    cnt = _degree_counts(senders, receivers)
    scnt = cnt[0, :N, None]
    rcnt = cnt[1, :N, None]

    packed = (senders | (receivers << 16)).reshape(NW, EPW)
    nfill = EPWP - EPW
    fill_r = N + jnp.arange(nfill, dtype=jnp.int32) % (NPADM - N)
    fill = jnp.broadcast_to(fill_r << 16, (NW, nfill))
    pidx2 = jnp.concatenate([packed, fill], axis=1)

    def ws(i):
        return [mlp_W1[i], mlp_b1[i][None], ln1_scale[i][None],
                ln1_bias[i][None], mlp_W2[i], mlp_b2[i][None],
                ln2_scale[i][None], ln2_bias[i][None], mlp_W3[i],
                mlp_b3[i][None]]

    m0 = _embed_mlp(nodes, W_embed, b_embed[None], ws(0), scnt)
    p0 = _messages(m0, pidx2)
    m1 = _mid(p0, m0, rcnt, ws(1), scnt)
    p1 = _messages(m1, pidx2)
    return _final(p1, m1, rcnt, W_dec, b_dec[None])


# speedup vs baseline: 3.4350x; 3.1821x over previous
import functools

import jax
import jax.numpy as jnp
from jax import lax
from jax.experimental import pallas as pl
from jax.experimental.pallas import tpu as pltpu
from jax.experimental.pallas import tpu_sc as plsc

N = 10000
E = 320000
D = 128
G = 16
NPG = N // G
NC = 2
NS = 16
NW = NC * NS
EPW = E // NW
EPT = E // NS
CH = 80
NCH = EPW // CH
NPAD = 10240
RED = NPAD // NS
NPADM = 10112
REDM = NPADM // NS


def _deg_body(sr_hbm, out_hbm, idx_v, cnt_v, redbuf, outbuf, shared):
    cid = lax.axis_index("c")
    sid = lax.axis_index("s")
    zeros = jnp.zeros((16,), jnp.float32)
    ones = jnp.ones((16,), jnp.float32)

    def zbody(i, _):
        cnt_v[pl.ds(i * 16, 16)] = zeros
        return 0
    lax.fori_loop(0, NPAD // 16, zbody, 0)

    pltpu.sync_copy(sr_hbm.at[pl.ds(cid * E + sid * EPT, EPT)], idx_v)

    def cbody(i, _):
        si = idx_v[pl.ds(i * 16, 16)]
        plsc.addupdate_scatter(cnt_v, [si], ones)
        return 0
    lax.fori_loop(0, EPT // 16, cbody, 0)

    pltpu.sync_copy(cnt_v, shared.at[sid])
    plsc.subcore_barrier()

    lo = sid * RED
    pltpu.sync_copy(shared.at[:, pl.ds(lo, RED)], redbuf)

    def rbody(i, _):
        acc = redbuf[0, pl.ds(i * 16, 16)]
        for t in range(1, NS):
            acc = acc + redbuf[t, pl.ds(i * 16, 16)]
        outbuf[pl.ds(i * 16, 16)] = acc
        return 0
    lax.fori_loop(0, RED // 16, rbody, 0)

    pltpu.sync_copy(outbuf, out_hbm.at[cid, pl.ds(lo, RED)])


def _degree_counts(senders, receivers):
    mesh = plsc.VectorSubcoreMesh(core_axis_name="c", subcore_axis_name="s")
    return pl.kernel(
        _deg_body,
        compiler_params=pltpu.CompilerParams(needs_layout_passes=False),
        out_type=jax.ShapeDtypeStruct((2, NPAD), jnp.float32),
        mesh=mesh,
        scratch_types=[
            pltpu.VMEM((EPT,), jnp.int32),
            pltpu.VMEM((NPAD,), jnp.float32),
            pltpu.VMEM((NS, RED), jnp.float32),
            pltpu.VMEM((RED,), jnp.float32),
            pltpu.VMEM_SHARED((NS, NPAD), jnp.float32),
        ],
    )(jnp.concatenate([senders, receivers]))


ZR = 32
NZC = REDM // ZR
ZREM = REDM - NZC * ZR


def _msg_body(m_hbm, pidx_hbm, out_hbm, pidx_v, sc0, rc0, sc1, rc1, sc2, rc2,
              rows0, rows1, rows2, zbuf, acc, g0, g1, g2, t0, t1, t2, z0):
    cid = lax.axis_index("c")
    sid = lax.axis_index("s")
    w = cid * NS + sid

    zeros = jnp.zeros((16,), jnp.float32)

    def zrow(r, _):
        for c in range(D // 16):
            zbuf[r, pl.ds(c * 16, 16)] = zeros
        return 0
    lax.fori_loop(0, ZR, zrow, 0)

    base = sid * REDM

    def zcopy(j, _):
        pltpu.async_copy(zbuf, acc.at[pl.ds(base + j * ZR, ZR)], z0)
        return 0
    lax.fori_loop(0, NZC, zcopy, 0)
    pltpu.async_copy(zbuf.at[pl.ds(0, ZREM)],
                     acc.at[pl.ds(base + NZC * ZR, ZREM)], z0)

    pltpu.sync_copy(pidx_hbm.at[w], pidx_v)

    scs = [sc0, sc1, sc2]
    rcs = [rc0, rc1, rc2]
    rows = [rows0, rows1, rows2]
    gs = [g0, g1, g2]
    ts = [t0, t1, t2]

    def unpack(i, b):
        def ub(j, _):
            v = pidx_v[pl.ds(i * CH + j * 16, 16)]
            scs[b][pl.ds(j * 16, 16)] = v & 0xFFFF
            rcs[b][pl.ds(j * 16, 16)] = v >> 16
            return 0
        lax.fori_loop(0, CH // 16, ub, 0)

    def start_gather(i, b):
        unpack(i, b)
        pltpu.async_copy(m_hbm.at[scs[b]], rows[b], gs[b])

    def wait_gather(b):
        pltpu.make_async_copy(m_hbm.at[scs[b]], rows[b], gs[b]).wait()

    def start_scatter(b):
        pltpu.async_copy(rows[b], acc.at[rcs[b]], ts[b], add=True)

    def wait_scatter(b):
        pltpu.make_async_copy(rows[b], acc.at[rcs[b]], ts[b]).wait()

    start_gather(0, 0)
    start_gather(1, 1)
    start_gather(2, 2)

    def zdrain(j, _):
        pltpu.make_async_copy(zbuf, acc.at[pl.ds(base + j * ZR, ZR)],
                              z0).wait()
        return 0
    lax.fori_loop(0, NZC, zdrain, 0)
    pltpu.make_async_copy(zbuf.at[pl.ds(0, ZREM)],
                          acc.at[pl.ds(base + NZC * ZR, ZREM)], z0).wait()
    plsc.subcore_barrier()

    wait_gather(0)
    start_scatter(0)

    def pbody(q, _):
        for j in range(3):
            c = 3 * q + 1 + j
            b = (1 + j) % 3
            wait_gather(b)
            start_scatter(b)
            wait_scatter(j)
            if j < 2:
                start_gather(c + 2, j)
            else:
                @pl.when(q < (NCH - 4) // 3)
                def _():
                    start_gather(c + 2, j)
        return 0
    lax.fori_loop(0, (NCH - 2) // 3, pbody, 0)

    wait_gather(1)
    start_scatter(1)
    wait_scatter(0)
    wait_scatter(1)

    plsc.subcore_barrier()
    pltpu.sync_copy(acc.at[pl.ds(sid * REDM, REDM)],
                    out_hbm.at[cid, pl.ds(sid * REDM, REDM)])


def _messages(m, pidx3):
    mesh = plsc.VectorSubcoreMesh(core_axis_name="c", subcore_axis_name="s")
    return pl.kernel(
        _msg_body,
        out_type=jax.ShapeDtypeStruct((NC, NPADM, D), jnp.float32),
        mesh=mesh,
        scratch_types=[
            pltpu.VMEM((EPW,), jnp.int32),
            pltpu.VMEM((CH,), jnp.int32),
            pltpu.VMEM((CH,), jnp.int32),
            pltpu.VMEM((CH,), jnp.int32),
            pltpu.VMEM((CH,), jnp.int32),
            pltpu.VMEM((CH,), jnp.int32),
            pltpu.VMEM((CH,), jnp.int32),
            pltpu.VMEM((CH, D), jnp.float32),
            pltpu.VMEM((CH, D), jnp.float32),
            pltpu.VMEM((CH, D), jnp.float32),
            pltpu.VMEM((ZR, D), jnp.float32),
            pltpu.VMEM_SHARED((NPADM, D), jnp.float32),
            pltpu.SemaphoreType.DMA,
            pltpu.SemaphoreType.DMA,
            pltpu.SemaphoreType.DMA,
            pltpu.SemaphoreType.DMA,
            pltpu.SemaphoreType.DMA,
            pltpu.SemaphoreType.DMA,
            pltpu.SemaphoreType.DMA,
        ],
    )(m, pidx3)


ROWS = 1000
NBLK = N // ROWS


def _layer_norm(x, scale, bias, eps=1e-6):
    mean = jnp.mean(x, axis=-1, keepdims=True)
    var = jnp.mean(jnp.square(x - mean), axis=-1, keepdims=True)
    return (x - mean) * lax.rsqrt(var + eps) * scale + bias


def _mlp_block(x, W1, b1, g1, be1, W2, b2, g2, be2, W3, b3):
    x = jax.nn.relu(jnp.dot(x, W1, preferred_element_type=jnp.float32) + b1)
    x = _layer_norm(x, g1, be1)
    x = jax.nn.relu(jnp.dot(x, W2, preferred_element_type=jnp.float32) + b2)
    x = _layer_norm(x, g2, be2)
    return jax.nn.relu(jnp.dot(x, W3, preferred_element_type=jnp.float32) + b3)


def _embed_mlp_body(nodes, We, be, W1, b1, g1, be1, W2, b2, g2, be2, W3, b3,
                    scnt, out):
    h = jnp.dot(nodes[...], We[...], preferred_element_type=jnp.float32) + be[...]
    x = _mlp_block(h, W1[...], b1[...], g1[...], be1[...], W2[...], b2[...],
                   g2[...], be2[...], W3[...], b3[...])
    out[...] = x * lax.rsqrt(scnt[...] + 1.0)


def _mid_body(p, m, rcnt, W1, b1, g1, be1, W2, b2, g2, be2, W3, b3,
              scnt, out):
    h = (p[0] + p[1] + m[...]) * lax.rsqrt(rcnt[...] + 1.0)
    x = _mlp_block(h, W1[...], b1[...], g1[...], be1[...], W2[...], b2[...],
                   g2[...], be2[...], W3[...], b3[...])
    out[...] = x * lax.rsqrt(scnt[...] + 1.0)


def _final_body(p, m, rcnt, Wd, bd, out):
    h = (p[0] + p[1] + m[...]) * lax.rsqrt(rcnt[...] + 1.0)
    gids = lax.broadcasted_iota(jnp.int32, (G, N), 0)
    nids = lax.broadcasted_iota(jnp.int32, (G, N), 1) // NPG
    seg = jnp.where(gids == nids, 1.0 / NPG, 0.0)
    pooled = jnp.dot(seg, h, preferred_element_type=jnp.float32)
    out[...] = jnp.dot(pooled, Wd[...], preferred_element_type=jnp.float32) + bd[...]


def _full(shape):
    return pl.BlockSpec(shape, lambda i: (0,) * len(shape))


def _wspecs():
    return [_full((D, D)), _full((1, D)), _full((1, D)), _full((1, D)),
            _full((D, D)), _full((1, D)), _full((1, D)), _full((1, D)),
            _full((D, D)), _full((1, D))]


def _embed_mlp(nodes, We, be, ws, scnt):
    return pl.pallas_call(
        _embed_mlp_body,
        grid=(NBLK,),
        in_specs=[pl.BlockSpec((ROWS, D), lambda i: (i, 0)),
                  _full((D, D)), _full((1, D)), *_wspecs(),
                  pl.BlockSpec((ROWS, 1), lambda i: (i, 0))],
        out_specs=pl.BlockSpec((ROWS, D), lambda i: (i, 0)),
        out_shape=jax.ShapeDtypeStruct((N, D), jnp.float32),
    )(nodes, We, be, *ws, scnt)


def _mid(p, m, rcnt, ws, scnt):
    return pl.pallas_call(
        _mid_body,
        grid=(NBLK,),
        in_specs=[pl.BlockSpec((NC, ROWS, D), lambda i: (0, i, 0)),
                  pl.BlockSpec((ROWS, D), lambda i: (i, 0)),
                  pl.BlockSpec((ROWS, 1), lambda i: (i, 0)),
                  *_wspecs(),
                  pl.BlockSpec((ROWS, 1), lambda i: (i, 0))],
        out_specs=pl.BlockSpec((ROWS, D), lambda i: (i, 0)),
        out_shape=jax.ShapeDtypeStruct((N, D), jnp.float32),
    )(p, m, rcnt, *ws, scnt)


def _final(p, m, rcnt, Wd, bd):
    return pl.pallas_call(
        _final_body,
        grid=(1,),
        in_specs=[pl.BlockSpec((NC, N, D), lambda i: (0, 0, 0)),
                  pl.BlockSpec((N, D), lambda i: (0, 0)),
                  pl.BlockSpec((N, 1), lambda i: (0, 0)),
                  pl.BlockSpec((D, D), lambda i: (0, 0)),
                  pl.BlockSpec((1, D), lambda i: (0, 0))],
        out_specs=pl.BlockSpec((G, D), lambda i: (0, 0)),
        out_shape=jax.ShapeDtypeStruct((G, D), jnp.float32),
    )(p, m, rcnt, Wd, bd)


def kernel(nodes, edges, senders, receivers, globals_, n_node, n_edge,
           W_embed, b_embed, mlp_W1, mlp_b1, ln1_scale, ln1_bias,
           mlp_W2, mlp_b2, ln2_scale, ln2_bias, mlp_W3, mlp_b3,
           W_dec, b_dec):
    senders = senders.astype(jnp.int32)
    receivers = receivers.astype(jnp.int32)

    cnt = _degree_counts(senders, receivers)
    scnt = cnt[0, :N, None]
    rcnt = cnt[1, :N, None]

    pidx3 = (senders | (receivers << 16)).reshape(NW, EPW)

    def ws(i):
        return [mlp_W1[i], mlp_b1[i][None], ln1_scale[i][None],
                ln1_bias[i][None], mlp_W2[i], mlp_b2[i][None],
                ln2_scale[i][None], ln2_bias[i][None], mlp_W3[i],
                mlp_b3[i][None]]

    m0 = _embed_mlp(nodes, W_embed, b_embed[None], ws(0), scnt)
    p0 = _messages(m0, pidx3)
    m1 = _mid(p0, m0, rcnt, ws(1), scnt)
    p1 = _messages(m1, pidx3)
    return _final(p1, m1, rcnt, W_dec, b_dec[None])


# speedup vs baseline: 3.4355x; 1.0001x over previous
import functools

import jax
import jax.numpy as jnp
from jax import lax
from jax.experimental import pallas as pl
from jax.experimental.pallas import tpu as pltpu
from jax.experimental.pallas import tpu_sc as plsc

N = 10000
E = 320000
D = 128
G = 16
NPG = N // G
NC = 2
NS = 16
NW = NC * NS
EPW = E // NW
EPT = E // NS
CH = 80
NCH = EPW // CH
NPAD = 10240
RED = NPAD // NS
NPADM = 10112
REDM = NPADM // NS


def _deg_body(sr_hbm, out_hbm, idx_v, cnt_v, redbuf, outbuf, shared):
    cid = lax.axis_index("c")
    sid = lax.axis_index("s")
    zeros = jnp.zeros((16,), jnp.float32)
    ones = jnp.ones((16,), jnp.float32)

    def zbody(i, _):
        cnt_v[pl.ds(i * 16, 16)] = zeros
        return 0
    lax.fori_loop(0, NPAD // 16, zbody, 0)

    pltpu.sync_copy(sr_hbm.at[pl.ds(cid * E + sid * EPT, EPT)], idx_v)

    def cbody(i, _):
        si = idx_v[pl.ds(i * 16, 16)]
        plsc.addupdate_scatter(cnt_v, [si], ones)
        return 0
    lax.fori_loop(0, EPT // 16, cbody, 0)

    pltpu.sync_copy(cnt_v, shared.at[sid])
    plsc.subcore_barrier()

    lo = sid * RED
    pltpu.sync_copy(shared.at[:, pl.ds(lo, RED)], redbuf)

    def rbody(i, _):
        acc = redbuf[0, pl.ds(i * 16, 16)]
        for t in range(1, NS):
            acc = acc + redbuf[t, pl.ds(i * 16, 16)]
        outbuf[pl.ds(i * 16, 16)] = acc
        return 0
    lax.fori_loop(0, RED // 16, rbody, 0)

    pltpu.sync_copy(outbuf, out_hbm.at[cid, pl.ds(lo, RED)])


def _degree_counts(senders, receivers):
    mesh = plsc.VectorSubcoreMesh(core_axis_name="c", subcore_axis_name="s")
    return pl.kernel(
        _deg_body,
        compiler_params=pltpu.CompilerParams(needs_layout_passes=False),
        out_type=jax.ShapeDtypeStruct((2, NPAD), jnp.float32),
        mesh=mesh,
        scratch_types=[
            pltpu.VMEM((EPT,), jnp.int32),
            pltpu.VMEM((NPAD,), jnp.float32),
            pltpu.VMEM((NS, RED), jnp.float32),
            pltpu.VMEM((RED,), jnp.float32),
            pltpu.VMEM_SHARED((NS, NPAD), jnp.float32),
        ],
    )(jnp.concatenate([senders, receivers]))


ZR = 32
NZC = REDM // ZR
ZREM = REDM - NZC * ZR


def _msg_body(m_hbm, pidx_hbm, out_hbm, pidx_v, sc0, rc0, sc1, rc1, sc2, rc2,
              rows0, rows1, rows2, zbuf, acc, g0, g1, g2, t0, t1, t2, z0):
    cid = lax.axis_index("c")
    sid = lax.axis_index("s")
    w = cid * NS + sid

    zeros = jnp.zeros((16,), jnp.float32)

    def zrow(r, _):
        for c in range(D // 16):
            zbuf[r, pl.ds(c * 16, 16)] = zeros
        return 0
    lax.fori_loop(0, ZR, zrow, 0)

    base = sid * REDM

    def zcopy(j, _):
        pltpu.async_copy(zbuf, acc.at[pl.ds(base + j * ZR, ZR)], z0)
        return 0
    lax.fori_loop(0, NZC, zcopy, 0)
    pltpu.async_copy(zbuf.at[pl.ds(0, ZREM)],
                     acc.at[pl.ds(base + NZC * ZR, ZREM)], z0)

    pltpu.sync_copy(pidx_hbm.at[w], pidx_v)

    scs = [sc0, sc1, sc2]
    rcs = [rc0, rc1, rc2]
    rows = [rows0, rows1, rows2]
    gs = [g0, g1, g2]
    ts = [t0, t1, t2]

    def unpack(i, b):
        def ub(j, _):
            v = pidx_v[pl.ds(i * CH + j * 16, 16)]
            scs[b][pl.ds(j * 16, 16)] = v & 0xFFFF
            rcs[b][pl.ds(j * 16, 16)] = v >> 16
            return 0
        lax.fori_loop(0, CH // 16, ub, 0)

    H = CH // 2

    def start_gather(i, b):
        unpack(i, b)
        pltpu.async_copy(m_hbm.at[scs[b].at[pl.ds(0, H)]],
                         rows[b].at[pl.ds(0, H)], gs[b])
        pltpu.async_copy(m_hbm.at[scs[b].at[pl.ds(H, H)]],
                         rows[b].at[pl.ds(H, H)], gs[b])

    def wait_gather(b):
        pltpu.make_async_copy(m_hbm.at[scs[b].at[pl.ds(0, H)]],
                              rows[b].at[pl.ds(0, H)], gs[b]).wait()
        pltpu.make_async_copy(m_hbm.at[scs[b].at[pl.ds(H, H)]],
                              rows[b].at[pl.ds(H, H)], gs[b]).wait()

    def start_scatter(b):
        pltpu.async_copy(rows[b], acc.at[rcs[b]], ts[b], add=True)

    def wait_scatter(b):
        pltpu.make_async_copy(rows[b], acc.at[rcs[b]], ts[b]).wait()

    start_gather(0, 0)
    start_gather(1, 1)
    start_gather(2, 2)

    def zdrain(j, _):
        pltpu.make_async_copy(zbuf, acc.at[pl.ds(base + j * ZR, ZR)],
                              z0).wait()
        return 0
    lax.fori_loop(0, NZC, zdrain, 0)
    pltpu.make_async_copy(zbuf.at[pl.ds(0, ZREM)],
                          acc.at[pl.ds(base + NZC * ZR, ZREM)], z0).wait()
    plsc.subcore_barrier()

    wait_gather(0)
    start_scatter(0)

    def pbody(q, _):
        for j in range(3):
            c = 3 * q + 1 + j
            b = (1 + j) % 3
            wait_gather(b)
            start_scatter(b)
            wait_scatter(j)
            if j < 2:
                start_gather(c + 2, j)
            else:
                @pl.when(q < (NCH - 4) // 3)
                def _():
                    start_gather(c + 2, j)
        return 0
    lax.fori_loop(0, (NCH - 2) // 3, pbody, 0)

    wait_gather(1)
    start_scatter(1)
    wait_scatter(0)
    wait_scatter(1)

    plsc.subcore_barrier()
    pltpu.sync_copy(acc.at[pl.ds(sid * REDM, REDM)],
                    out_hbm.at[cid, pl.ds(sid * REDM, REDM)])


def _messages(m, pidx3):
    mesh = plsc.VectorSubcoreMesh(core_axis_name="c", subcore_axis_name="s")
    return pl.kernel(
        _msg_body,
        out_type=jax.ShapeDtypeStruct((NC, NPADM, D), jnp.float32),
        mesh=mesh,
        scratch_types=[
            pltpu.VMEM((EPW,), jnp.int32),
            pltpu.VMEM((CH,), jnp.int32),
            pltpu.VMEM((CH,), jnp.int32),
            pltpu.VMEM((CH,), jnp.int32),
            pltpu.VMEM((CH,), jnp.int32),
            pltpu.VMEM((CH,), jnp.int32),
            pltpu.VMEM((CH,), jnp.int32),
            pltpu.VMEM((CH, D), jnp.float32),
            pltpu.VMEM((CH, D), jnp.float32),
            pltpu.VMEM((CH, D), jnp.float32),
            pltpu.VMEM((ZR, D), jnp.float32),
            pltpu.VMEM_SHARED((NPADM, D), jnp.float32),
            pltpu.SemaphoreType.DMA,
            pltpu.SemaphoreType.DMA,
            pltpu.SemaphoreType.DMA,
            pltpu.SemaphoreType.DMA,
            pltpu.SemaphoreType.DMA,
            pltpu.SemaphoreType.DMA,
            pltpu.SemaphoreType.DMA,
        ],
    )(m, pidx3)


ROWS = 1000
NBLK = N // ROWS


def _layer_norm(x, scale, bias, eps=1e-6):
    mean = jnp.mean(x, axis=-1, keepdims=True)
    var = jnp.mean(jnp.square(x - mean), axis=-1, keepdims=True)
    return (x - mean) * lax.rsqrt(var + eps) * scale + bias


def _mlp_block(x, W1, b1, g1, be1, W2, b2, g2, be2, W3, b3):
    x = jax.nn.relu(jnp.dot(x, W1, preferred_element_type=jnp.float32) + b1)
    x = _layer_norm(x, g1, be1)
    x = jax.nn.relu(jnp.dot(x, W2, preferred_element_type=jnp.float32) + b2)
    x = _layer_norm(x, g2, be2)
    return jax.nn.relu(jnp.dot(x, W3, preferred_element_type=jnp.float32) + b3)


def _embed_mlp_body(nodes, We, be, W1, b1, g1, be1, W2, b2, g2, be2, W3, b3,
                    scnt, out):
    h = jnp.dot(nodes[...], We[...], preferred_element_type=jnp.float32) + be[...]
    x = _mlp_block(h, W1[...], b1[...], g1[...], be1[...], W2[...], b2[...],
                   g2[...], be2[...], W3[...], b3[...])
    out[...] = x * lax.rsqrt(scnt[...] + 1.0)


def _mid_body(p, m, rcnt, W1, b1, g1, be1, W2, b2, g2, be2, W3, b3,
              scnt, out):
    h = (p[0] + p[1] + m[...]) * lax.rsqrt(rcnt[...] + 1.0)
    x = _mlp_block(h, W1[...], b1[...], g1[...], be1[...], W2[...], b2[...],
                   g2[...], be2[...], W3[...], b3[...])
    out[...] = x * lax.rsqrt(scnt[...] + 1.0)


def _final_body(p, m, rcnt, Wd, bd, out):
    h = (p[0] + p[1] + m[...]) * lax.rsqrt(rcnt[...] + 1.0)
    gids = lax.broadcasted_iota(jnp.int32, (G, N), 0)
    nids = lax.broadcasted_iota(jnp.int32, (G, N), 1) // NPG
    seg = jnp.where(gids == nids, 1.0 / NPG, 0.0)
    pooled = jnp.dot(seg, h, preferred_element_type=jnp.float32)
    out[...] = jnp.dot(pooled, Wd[...], preferred_element_type=jnp.float32) + bd[...]


def _full(shape):
    return pl.BlockSpec(shape, lambda i: (0,) * len(shape))


def _wspecs():
    return [_full((D, D)), _full((1, D)), _full((1, D)), _full((1, D)),
            _full((D, D)), _full((1, D)), _full((1, D)), _full((1, D)),
            _full((D, D)), _full((1, D))]


def _embed_mlp(nodes, We, be, ws, scnt):
    return pl.pallas_call(
        _embed_mlp_body,
        grid=(NBLK,),
        in_specs=[pl.BlockSpec((ROWS, D), lambda i: (i, 0)),
                  _full((D, D)), _full((1, D)), *_wspecs(),
                  pl.BlockSpec((ROWS, 1), lambda i: (i, 0))],
        out_specs=pl.BlockSpec((ROWS, D), lambda i: (i, 0)),
        out_shape=jax.ShapeDtypeStruct((N, D), jnp.float32),
    )(nodes, We, be, *ws, scnt)


def _mid(p, m, rcnt, ws, scnt):
    return pl.pallas_call(
        _mid_body,
        grid=(NBLK,),
        in_specs=[pl.BlockSpec((NC, ROWS, D), lambda i: (0, i, 0)),
                  pl.BlockSpec((ROWS, D), lambda i: (i, 0)),
                  pl.BlockSpec((ROWS, 1), lambda i: (i, 0)),
                  *_wspecs(),
                  pl.BlockSpec((ROWS, 1), lambda i: (i, 0))],
        out_specs=pl.BlockSpec((ROWS, D), lambda i: (i, 0)),
        out_shape=jax.ShapeDtypeStruct((N, D), jnp.float32),
    )(p, m, rcnt, *ws, scnt)


def _final(p, m, rcnt, Wd, bd):
    return pl.pallas_call(
        _final_body,
        grid=(1,),
        in_specs=[pl.BlockSpec((NC, N, D), lambda i: (0, 0, 0)),
                  pl.BlockSpec((N, D), lambda i: (0, 0)),
                  pl.BlockSpec((N, 1), lambda i: (0, 0)),
                  pl.BlockSpec((D, D), lambda i: (0, 0)),
                  pl.BlockSpec((1, D), lambda i: (0, 0))],
        out_specs=pl.BlockSpec((G, D), lambda i: (0, 0)),
        out_shape=jax.ShapeDtypeStruct((G, D), jnp.float32),
    )(p, m, rcnt, Wd, bd)


def kernel(nodes, edges, senders, receivers, globals_, n_node, n_edge,
           W_embed, b_embed, mlp_W1, mlp_b1, ln1_scale, ln1_bias,
           mlp_W2, mlp_b2, ln2_scale, ln2_bias, mlp_W3, mlp_b3,
           W_dec, b_dec):
    senders = senders.astype(jnp.int32)
    receivers = receivers.astype(jnp.int32)

    cnt = _degree_counts(senders, receivers)
    scnt = cnt[0, :N, None]
    rcnt = cnt[1, :N, None]

    pidx3 = (senders | (receivers << 16)).reshape(NW, EPW)

    def ws(i):
        return [mlp_W1[i], mlp_b1[i][None], ln1_scale[i][None],
                ln1_bias[i][None], mlp_W2[i], mlp_b2[i][None],
                ln2_scale[i][None], ln2_bias[i][None], mlp_W3[i],
                mlp_b3[i][None]]

    m0 = _embed_mlp(nodes, W_embed, b_embed[None], ws(0), scnt)
    p0 = _messages(m0, pidx3)
    m1 = _mid(p0, m0, rcnt, ws(1), scnt)
    p1 = _messages(m1, pidx3)
    return _final(p1, m1, rcnt, W_dec, b_dec[None])


# speedup vs baseline: 3.4623x; 1.0078x over previous
import functools

import jax
import jax.numpy as jnp
from jax import lax
from jax.experimental import pallas as pl
from jax.experimental.pallas import tpu as pltpu
from jax.experimental.pallas import tpu_sc as plsc

N = 10000
E = 320000
D = 128
G = 16
NPG = N // G
NC = 2
NS = 16
NW = NC * NS
EPW = E // NW
EPT = E // NS
CH = 80
NCH = EPW // CH
NPAD = 10240
RED = NPAD // NS
NPADM = 10112
REDM = NPADM // NS


def _deg_body(sr_hbm, out_hbm, idx_v, cnt_v, redbuf, outbuf, shared, dsem):
    cid = lax.axis_index("c")
    sid = lax.axis_index("s")
    zeros = jnp.zeros((16,), jnp.float32)
    ones = jnp.ones((16,), jnp.float32)

    cp = pltpu.async_copy(sr_hbm.at[pl.ds(cid * E + sid * EPT, EPT)],
                          idx_v, dsem)

    def zbody(i, _):
        cnt_v[pl.ds(i * 16, 16)] = zeros
        return 0
    lax.fori_loop(0, NPAD // 16, zbody, 0)
    cp.wait()

    def cbody(i, _):
        for k in range(4):
            si = idx_v[pl.ds(i * 64 + k * 16, 16)]
            plsc.addupdate_scatter(cnt_v, [si], ones)
        return 0
    lax.fori_loop(0, EPT // 64, cbody, 0)

    pltpu.sync_copy(cnt_v, shared.at[sid])
    plsc.subcore_barrier()

    lo = sid * RED
    pltpu.sync_copy(shared.at[:, pl.ds(lo, RED)], redbuf)

    def rbody(i, _):
        acc = redbuf[0, pl.ds(i * 16, 16)]
        for t in range(1, NS):
            acc = acc + redbuf[t, pl.ds(i * 16, 16)]
        outbuf[pl.ds(i * 16, 16)] = acc
        return 0
    lax.fori_loop(0, RED // 16, rbody, 0)

    pltpu.sync_copy(outbuf, out_hbm.at[cid, pl.ds(lo, RED)])


def _degree_counts(senders, receivers):
    mesh = plsc.VectorSubcoreMesh(core_axis_name="c", subcore_axis_name="s")
    return pl.kernel(
        _deg_body,
        compiler_params=pltpu.CompilerParams(needs_layout_passes=False),
        out_type=jax.ShapeDtypeStruct((2, NPAD), jnp.float32),
        mesh=mesh,
        scratch_types=[
            pltpu.VMEM((EPT,), jnp.int32),
            pltpu.VMEM((NPAD,), jnp.float32),
            pltpu.VMEM((NS, RED), jnp.float32),
            pltpu.VMEM((RED,), jnp.float32),
            pltpu.VMEM_SHARED((NS, NPAD), jnp.float32),
            pltpu.SemaphoreType.DMA,
        ],
    )(jnp.concatenate([senders, receivers]))


ZR = 32
NZC = REDM // ZR
ZREM = REDM - NZC * ZR


def _msg_body(m_hbm, pidx_hbm, out_hbm, pidx_v, sc0, rc0, sc1, rc1, sc2, rc2,
              rows0, rows1, rows2, zbuf, acc, g0, g1, g2, t0, t1, t2, z0):
    cid = lax.axis_index("c")
    sid = lax.axis_index("s")
    w = cid * NS + sid

    zeros = jnp.zeros((16,), jnp.float32)

    def zrow(r, _):
        for c in range(D // 16):
            zbuf[r, pl.ds(c * 16, 16)] = zeros
        return 0
    lax.fori_loop(0, ZR, zrow, 0)

    base = sid * REDM

    def zcopy(j, _):
        pltpu.async_copy(zbuf, acc.at[pl.ds(base + j * ZR, ZR)], z0)
        return 0
    lax.fori_loop(0, NZC, zcopy, 0)
    pltpu.async_copy(zbuf.at[pl.ds(0, ZREM)],
                     acc.at[pl.ds(base + NZC * ZR, ZREM)], z0)

    pltpu.sync_copy(pidx_hbm.at[w], pidx_v)

    scs = [sc0, sc1, sc2]
    rcs = [rc0, rc1, rc2]
    rows = [rows0, rows1, rows2]
    gs = [g0, g1, g2]
    ts = [t0, t1, t2]

    def unpack(i, b):
        def ub(j, _):
            v = pidx_v[pl.ds(i * CH + j * 16, 16)]
            scs[b][pl.ds(j * 16, 16)] = v & 0xFFFF
            rcs[b][pl.ds(j * 16, 16)] = v >> 16
            return 0
        lax.fori_loop(0, CH // 16, ub, 0)

    def start_gather(i, b):
        unpack(i, b)
        pltpu.async_copy(m_hbm.at[scs[b]], rows[b], gs[b])

    def wait_gather(b):
        pltpu.make_async_copy(m_hbm.at[scs[b]], rows[b], gs[b]).wait()

    def start_scatter(b):
        pltpu.async_copy(rows[b], acc.at[rcs[b]], ts[b], add=True)

    def wait_scatter(b):
        pltpu.make_async_copy(rows[b], acc.at[rcs[b]], ts[b]).wait()

    start_gather(0, 0)
    start_gather(1, 1)
    start_gather(2, 2)

    def zdrain(j, _):
        pltpu.make_async_copy(zbuf, acc.at[pl.ds(base + j * ZR, ZR)],
                              z0).wait()
        return 0
    lax.fori_loop(0, NZC, zdrain, 0)
    pltpu.make_async_copy(zbuf.at[pl.ds(0, ZREM)],
                          acc.at[pl.ds(base + NZC * ZR, ZREM)], z0).wait()
    plsc.subcore_barrier()

    wait_gather(0)
    start_scatter(0)

    def pbody(q, _):
        for j in range(3):
            c = 3 * q + 1 + j
            b = (1 + j) % 3
            wait_gather(b)
            start_scatter(b)
            wait_scatter(j)
            if j < 2:
                start_gather(c + 2, j)
            else:
                @pl.when(q < (NCH - 4) // 3)
                def _():
                    start_gather(c + 2, j)
        return 0
    lax.fori_loop(0, (NCH - 2) // 3, pbody, 0)

    wait_gather(1)
    start_scatter(1)
    wait_scatter(0)
    wait_scatter(1)

    plsc.subcore_barrier()
    pltpu.sync_copy(acc.at[pl.ds(sid * REDM, REDM)],
                    out_hbm.at[cid, pl.ds(sid * REDM, REDM)])


def _messages(m, pidx3):
    mesh = plsc.VectorSubcoreMesh(core_axis_name="c", subcore_axis_name="s")
    return pl.kernel(
        _msg_body,
        out_type=jax.ShapeDtypeStruct((NC, NPADM, D), jnp.float32),
        mesh=mesh,
        scratch_types=[
            pltpu.VMEM((EPW,), jnp.int32),
            pltpu.VMEM((CH,), jnp.int32),
            pltpu.VMEM((CH,), jnp.int32),
            pltpu.VMEM((CH,), jnp.int32),
            pltpu.VMEM((CH,), jnp.int32),
            pltpu.VMEM((CH,), jnp.int32),
            pltpu.VMEM((CH,), jnp.int32),
            pltpu.VMEM((CH, D), jnp.float32),
            pltpu.VMEM((CH, D), jnp.float32),
            pltpu.VMEM((CH, D), jnp.float32),
            pltpu.VMEM((ZR, D), jnp.float32),
            pltpu.VMEM_SHARED((NPADM, D), jnp.float32),
            pltpu.SemaphoreType.DMA,
            pltpu.SemaphoreType.DMA,
            pltpu.SemaphoreType.DMA,
            pltpu.SemaphoreType.DMA,
            pltpu.SemaphoreType.DMA,
            pltpu.SemaphoreType.DMA,
            pltpu.SemaphoreType.DMA,
        ],
    )(m, pidx3)


ROWS = 1000
NBLK = N // ROWS


def _layer_norm(x, scale, bias, eps=1e-6):
    mean = jnp.mean(x, axis=-1, keepdims=True)
    var = jnp.mean(jnp.square(x - mean), axis=-1, keepdims=True)
    return (x - mean) * lax.rsqrt(var + eps) * scale + bias


def _mlp_block(x, W1, b1, g1, be1, W2, b2, g2, be2, W3, b3):
    x = jax.nn.relu(jnp.dot(x, W1, preferred_element_type=jnp.float32) + b1)
    x = _layer_norm(x, g1, be1)
    x = jax.nn.relu(jnp.dot(x, W2, preferred_element_type=jnp.float32) + b2)
    x = _layer_norm(x, g2, be2)
    return jax.nn.relu(jnp.dot(x, W3, preferred_element_type=jnp.float32) + b3)


def _embed_mlp_body(nodes, We, be, W1, b1, g1, be1, W2, b2, g2, be2, W3, b3,
                    scnt, out):
    h = jnp.dot(nodes[...], We[...], preferred_element_type=jnp.float32) + be[...]
    x = _mlp_block(h, W1[...], b1[...], g1[...], be1[...], W2[...], b2[...],
                   g2[...], be2[...], W3[...], b3[...])
    out[...] = x * lax.rsqrt(scnt[...] + 1.0)


def _mid_body(p, m, rcnt, W1, b1, g1, be1, W2, b2, g2, be2, W3, b3,
              scnt, out):
    h = (p[0] + p[1] + m[...]) * lax.rsqrt(rcnt[...] + 1.0)
    x = _mlp_block(h, W1[...], b1[...], g1[...], be1[...], W2[...], b2[...],
                   g2[...], be2[...], W3[...], b3[...])
    out[...] = x * lax.rsqrt(scnt[...] + 1.0)


def _final_body(p, m, rcnt, Wd, bd, out):
    h = (p[0] + p[1] + m[...]) * lax.rsqrt(rcnt[...] + 1.0)
    gids = lax.broadcasted_iota(jnp.int32, (G, N), 0)
    nids = lax.broadcasted_iota(jnp.int32, (G, N), 1) // NPG
    seg = jnp.where(gids == nids, 1.0 / NPG, 0.0)
    pooled = jnp.dot(seg, h, preferred_element_type=jnp.float32)
    out[...] = jnp.dot(pooled, Wd[...], preferred_element_type=jnp.float32) + bd[...]


def _full(shape):
    return pl.BlockSpec(shape, lambda i: (0,) * len(shape))


def _wspecs():
    return [_full((D, D)), _full((1, D)), _full((1, D)), _full((1, D)),
            _full((D, D)), _full((1, D)), _full((1, D)), _full((1, D)),
            _full((D, D)), _full((1, D))]


def _embed_mlp(nodes, We, be, ws, scnt):
    return pl.pallas_call(
        _embed_mlp_body,
        grid=(NBLK,),
        in_specs=[pl.BlockSpec((ROWS, D), lambda i: (i, 0)),
                  _full((D, D)), _full((1, D)), *_wspecs(),
                  pl.BlockSpec((ROWS, 1), lambda i: (i, 0))],
        out_specs=pl.BlockSpec((ROWS, D), lambda i: (i, 0)),
        out_shape=jax.ShapeDtypeStruct((N, D), jnp.float32),
    )(nodes, We, be, *ws, scnt)


def _mid(p, m, rcnt, ws, scnt):
    return pl.pallas_call(
        _mid_body,
        grid=(NBLK,),
        in_specs=[pl.BlockSpec((NC, ROWS, D), lambda i: (0, i, 0)),
                  pl.BlockSpec((ROWS, D), lambda i: (i, 0)),
                  pl.BlockSpec((ROWS, 1), lambda i: (i, 0)),
                  *_wspecs(),
                  pl.BlockSpec((ROWS, 1), lambda i: (i, 0))],
        out_specs=pl.BlockSpec((ROWS, D), lambda i: (i, 0)),
        out_shape=jax.ShapeDtypeStruct((N, D), jnp.float32),
    )(p, m, rcnt, *ws, scnt)


def _final(p, m, rcnt, Wd, bd):
    return pl.pallas_call(
        _final_body,
        grid=(1,),
        in_specs=[pl.BlockSpec((NC, N, D), lambda i: (0, 0, 0)),
                  pl.BlockSpec((N, D), lambda i: (0, 0)),
                  pl.BlockSpec((N, 1), lambda i: (0, 0)),
                  pl.BlockSpec((D, D), lambda i: (0, 0)),
                  pl.BlockSpec((1, D), lambda i: (0, 0))],
        out_specs=pl.BlockSpec((G, D), lambda i: (0, 0)),
        out_shape=jax.ShapeDtypeStruct((G, D), jnp.float32),
    )(p, m, rcnt, Wd, bd)


def kernel(nodes, edges, senders, receivers, globals_, n_node, n_edge,
           W_embed, b_embed, mlp_W1, mlp_b1, ln1_scale, ln1_bias,
           mlp_W2, mlp_b2, ln2_scale, ln2_bias, mlp_W3, mlp_b3,
           W_dec, b_dec):
    senders = senders.astype(jnp.int32)
    receivers = receivers.astype(jnp.int32)

    cnt = _degree_counts(senders, receivers)
    scnt = cnt[0, :N, None]
    rcnt = cnt[1, :N, None]

    pidx3 = (senders | (receivers << 16)).reshape(NW, EPW)

    def ws(i):
        return [mlp_W1[i], mlp_b1[i][None], ln1_scale[i][None],
                ln1_bias[i][None], mlp_W2[i], mlp_b2[i][None],
                ln2_scale[i][None], ln2_bias[i][None], mlp_W3[i],
                mlp_b3[i][None]]

    m0 = _embed_mlp(nodes, W_embed, b_embed[None], ws(0), scnt)
    p0 = _messages(m0, pidx3)
    m1 = _mid(p0, m0, rcnt, ws(1), scnt)
    p1 = _messages(m1, pidx3)
    return _final(p1, m1, rcnt, W_dec, b_dec[None])
